# baseline (speedup 1.0000x reference)
import functools

import jax
import jax.numpy as jnp
from jax import lax
from jax.experimental import pallas as pl
from jax.experimental.pallas import tpu as pltpu
from jax.experimental.pallas import tpu_sc as plsc

N = 10000
E = 320000
D = 128
DH = D // 2
NC = 2
NS = 16
K = 256
CPT = 80
EPT = CPT * K
E_PAD = NS * EPT
NBUF = 2
N_PAD = 10240
RPT = N_PAD // NS
RCH = 80


def _sc_segsum(h2, srcx, dstx):
    mesh = plsc.VectorSubcoreMesh(core_axis_name="c", subcore_axis_name="s")

    @functools.partial(
        pl.kernel,
        out_type=jax.ShapeDtypeStruct((NC * N, DH), jnp.float32),
        mesh=mesh,
        scratch_types=[
            pltpu.VMEM((CPT, K), jnp.int32),
            pltpu.VMEM((CPT, K), jnp.int32),
            pltpu.VMEM((NBUF, K, DH), jnp.float32),
            pltpu.VMEM((RCH, DH), jnp.float32),
            pltpu.VMEM_SHARED((N_PAD, DH), jnp.float32),
            [pltpu.SemaphoreType.DMA] * NBUF,
        ],
        compiler_params=pltpu.CompilerParams(use_tc_tiling_on_sc=False),
    )
    def seg_kernel(h_hbm, src_hbm, dst_hbm, out_hbm,
                   src_v, dst_v, rows_v, buf_v, acc_sh, gsems):
        cid = lax.axis_index("c")
        sid = lax.axis_index("s")

        zeros16 = jnp.zeros((16,), jnp.float32)

        def zrow(i, carry):
            for c16 in range(DH // 16):
                buf_v[i, pl.ds(c16 * 16, 16)] = zeros16
            return carry

        lax.fori_loop(0, RCH, zrow, 0)
        row0 = sid * RPT
        for r in range(RPT // RCH):
            pltpu.sync_copy(buf_v, acc_sh.at[pl.ds(row0 + r * RCH, RCH)])
        plsc.subcore_barrier()

        pltpu.sync_copy(src_hbm.at[cid * NS + sid], src_v)
        pltpu.sync_copy(dst_hbm.at[sid], dst_v)

        for b in range(NBUF):
            pltpu.async_copy(h_hbm.at[src_v.at[b]], rows_v.at[b], gsems[b])

        def step(j, b):
            pltpu.make_async_copy(h_hbm.at[src_v.at[j]], rows_v.at[b], gsems[b]).wait()
            pltpu.sync_copy(rows_v.at[b], acc_sh.at[dst_v.at[j]], add=True)

        def outer(i, carry):
            for b in range(NBUF):
                j = i * NBUF + b
                step(j, b)
                pltpu.async_copy(h_hbm.at[src_v.at[j + NBUF]], rows_v.at[b], gsems[b])
            return carry

        lax.fori_loop(0, (CPT - NBUF) // NBUF, outer, 0)
        for b in range(NBUF):
            step(CPT - NBUF + b, b)
        plsc.subcore_barrier()

        nch = jnp.minimum(RPT, N - row0) // RCH

        def wb(r, carry):
            r0 = row0 + r * RCH
            pltpu.sync_copy(acc_sh.at[pl.ds(r0, RCH)], buf_v)
            pltpu.sync_copy(buf_v, out_hbm.at[pl.ds(cid * N + r0, RCH)])
            return carry

        lax.fori_loop(0, nch, wb, 0)

    return seg_kernel(h2, srcx, dstx)


R = 1000


def _agg_specs():
    return [
        pl.BlockSpec((R, DH), lambda i: (i, 0)),
        pl.BlockSpec((R, DH), lambda i: (N // R + i, 0)),
    ]


def _w_specs():
    return [
        pl.BlockSpec((D, D), lambda i: (0, 0)),
        pl.BlockSpec((1, D), lambda i: (0, 0)),
        pl.BlockSpec((D, D), lambda i: (0, 0)),
        pl.BlockSpec((1, D), lambda i: (0, 0)),
    ]


def _mlp(x, scale, wa, ba, wb, bb, relu_out):
    z = jnp.dot(x * scale, wa, preferred_element_type=jnp.float32) + ba
    z = jnp.maximum(z, 0.0)
    o = jnp.dot(z, wb, preferred_element_type=jnp.float32) + bb
    return jnp.maximum(o, 0.0) if relu_out else o


def _tc_layer1(x, agg, scale, Wa, ba, Wb, bb):
    def body(scale_ref, h_ref, a0_ref, a1_ref, wa_ref, ba_ref, wb_ref, bb_ref, o_ref):
        agg_blk = jnp.concatenate([a0_ref[...], a1_ref[...]], axis=1)
        z = h_ref[...] * scale_ref[0] + agg_blk
        o = _mlp(z, 1.0, wa_ref[...], ba_ref[...], wb_ref[...], bb_ref[...], True)
        o_ref[0] = o[:, :DH]
        o_ref[1] = o[:, DH:]

    return pl.pallas_call(
        body,
        grid=(N // R,),
        in_specs=[
            pl.BlockSpec(memory_space=pltpu.SMEM),
            pl.BlockSpec((R, D), lambda i: (i, 0)),
            *_agg_specs(),
            *_w_specs(),
        ],
        out_specs=pl.BlockSpec((NC, R, DH), lambda i: (0, i, 0)),
        out_shape=jax.ShapeDtypeStruct((NC, N, DH), jnp.float32),
    )(scale, x, agg, agg, Wa, ba.reshape(1, D), Wb, bb.reshape(1, D))


def _tc_layer2(h2, agg, scale, Wa, ba, Wb, bb):
    def body(scale_ref, h_ref, a0_ref, a1_ref, wa_ref, ba_ref, wb_ref, bb_ref, o_ref):
        h_blk = jnp.concatenate([h_ref[0], h_ref[1]], axis=1)
        agg_blk = jnp.concatenate([a0_ref[...], a1_ref[...]], axis=1)
        z = h_blk * scale_ref[0] + agg_blk
        o_ref[...] = _mlp(z, 1.0, wa_ref[...], ba_ref[...], wb_ref[...], bb_ref[...], False)

    return pl.pallas_call(
        body,
        grid=(N // R,),
        in_specs=[
            pl.BlockSpec(memory_space=pltpu.SMEM),
            pl.BlockSpec((NC, R, DH), lambda i: (0, i, 0)),
            *_agg_specs(),
            *_w_specs(),
        ],
        out_specs=pl.BlockSpec((R, D), lambda i: (i, 0)),
        out_shape=jax.ShapeDtypeStruct((N, D), jnp.float32),
    )(scale, h2, agg, agg, Wa, ba.reshape(1, D), Wb, bb.reshape(1, D))


def kernel(x, edge_index, eps1, W1a, b1a, W1b, b1b, eps2, W2a, b2a, W2b, b2b):
    src = jnp.pad(edge_index[0], (0, E_PAD - E)).reshape(NS, CPT, K)
    dst = jnp.pad(edge_index[1], (0, E_PAD - E),
                  constant_values=N).reshape(NS, CPT, K)
    srcx = jnp.concatenate([src, src + N], axis=0)
    s1 = (1.0 + eps1).reshape(1)
    s2 = (1.0 + eps2).reshape(1)

    x2 = jnp.concatenate([x[:, :DH], x[:, DH:]], axis=0)
    agg1 = _sc_segsum(x2, srcx, dst)
    h2 = _tc_layer1(x, agg1, s1, W1a, b1a, W1b, b1b)
    agg2 = _sc_segsum(h2.reshape(NC * N, DH), srcx, dst)
    return _tc_layer2(h2, agg2, s2, W2a, b2a, W2b, b2b)


# speedup vs baseline: 1.3037x; 1.3037x over previous
import functools

import jax
import jax.numpy as jnp
from jax import lax
from jax.experimental import pallas as pl
from jax.experimental.pallas import tpu as pltpu
from jax.experimental.pallas import tpu_sc as plsc

N = 10000
E = 320000
D = 128
DH = D // 2
NC = 2
NS = 16
K = 64
CPT = 314
EPT = CPT * K
E_PAD = NS * EPT
NBUF = 2
N_PAD = 10240
RPT = N_PAD // NS
RCH = 80


def _sc_segsum(h2, srcx, dstx):
    mesh = plsc.VectorSubcoreMesh(core_axis_name="c", subcore_axis_name="s")

    @functools.partial(
        pl.kernel,
        out_type=jax.ShapeDtypeStruct((NC * N, DH), jnp.float32),
        mesh=mesh,
        scratch_types=[
            pltpu.VMEM((CPT, K), jnp.int32),
            pltpu.VMEM((CPT, K), jnp.int32),
            pltpu.VMEM((NBUF, K, DH), jnp.float32),
            pltpu.VMEM((RCH, DH), jnp.float32),
            pltpu.VMEM_SHARED((N_PAD, DH), jnp.float32),
            [pltpu.SemaphoreType.DMA] * NBUF,
        ],
        compiler_params=pltpu.CompilerParams(use_tc_tiling_on_sc=False),
    )
    def seg_kernel(h_hbm, src_hbm, dst_hbm, out_hbm,
                   src_v, dst_v, rows_v, buf_v, acc_sh, gsems):
        cid = lax.axis_index("c")
        sid = lax.axis_index("s")

        zeros16 = jnp.zeros((16,), jnp.float32)

        def zrow(i, carry):
            for c16 in range(DH // 16):
                buf_v[i, pl.ds(c16 * 16, 16)] = zeros16
            return carry

        lax.fori_loop(0, RCH, zrow, 0)
        row0 = sid * RPT
        for r in range(RPT // RCH):
            pltpu.sync_copy(buf_v, acc_sh.at[pl.ds(row0 + r * RCH, RCH)])
        plsc.subcore_barrier()

        pltpu.sync_copy(src_hbm.at[cid * NS + sid], src_v)
        pltpu.sync_copy(dst_hbm.at[sid], dst_v)

        for b in range(NBUF):
            pltpu.async_copy(h_hbm.at[src_v.at[b]], rows_v.at[b], gsems[b])

        def step(j, b):
            pltpu.make_async_copy(h_hbm.at[src_v.at[j]], rows_v.at[b], gsems[b]).wait()
            pltpu.sync_copy(rows_v.at[b], acc_sh.at[dst_v.at[j]], add=True)

        def outer(i, carry):
            for b in range(NBUF):
                j = i * NBUF + b
                step(j, b)
                pltpu.async_copy(h_hbm.at[src_v.at[j + NBUF]], rows_v.at[b], gsems[b])
            return carry

        lax.fori_loop(0, (CPT - NBUF) // NBUF, outer, 0)
        for b in range(NBUF):
            step(CPT - NBUF + b, b)
        plsc.subcore_barrier()

        nch = jnp.minimum(RPT, N - row0) // RCH

        def wb(r, carry):
            r0 = row0 + r * RCH
            pltpu.sync_copy(acc_sh.at[pl.ds(r0, RCH)], buf_v)
            pltpu.sync_copy(buf_v, out_hbm.at[pl.ds(cid * N + r0, RCH)])
            return carry

        lax.fori_loop(0, nch, wb, 0)

    return seg_kernel(h2, srcx, dstx)


R = 1000


def _agg_specs():
    return [
        pl.BlockSpec((R, DH), lambda i: (i, 0)),
        pl.BlockSpec((R, DH), lambda i: (N // R + i, 0)),
    ]


def _w_specs():
    return [
        pl.BlockSpec((D, D), lambda i: (0, 0)),
        pl.BlockSpec((1, D), lambda i: (0, 0)),
        pl.BlockSpec((D, D), lambda i: (0, 0)),
        pl.BlockSpec((1, D), lambda i: (0, 0)),
    ]


def _mlp(x, scale, wa, ba, wb, bb, relu_out):
    z = jnp.dot(x * scale, wa, preferred_element_type=jnp.float32) + ba
    z = jnp.maximum(z, 0.0)
    o = jnp.dot(z, wb, preferred_element_type=jnp.float32) + bb
    return jnp.maximum(o, 0.0) if relu_out else o


def _tc_layer1(x, agg, scale, Wa, ba, Wb, bb):
    def body(scale_ref, h_ref, a0_ref, a1_ref, wa_ref, ba_ref, wb_ref, bb_ref, o_ref):
        agg_blk = jnp.concatenate([a0_ref[...], a1_ref[...]], axis=1)
        z = h_ref[...] * scale_ref[0] + agg_blk
        o = _mlp(z, 1.0, wa_ref[...], ba_ref[...], wb_ref[...], bb_ref[...], True)
        o_ref[0] = o[:, :DH]
        o_ref[1] = o[:, DH:]

    return pl.pallas_call(
        body,
        grid=(N // R,),
        in_specs=[
            pl.BlockSpec(memory_space=pltpu.SMEM),
            pl.BlockSpec((R, D), lambda i: (i, 0)),
            *_agg_specs(),
            *_w_specs(),
        ],
        out_specs=pl.BlockSpec((NC, R, DH), lambda i: (0, i, 0)),
        out_shape=jax.ShapeDtypeStruct((NC, N, DH), jnp.float32),
    )(scale, x, agg, agg, Wa, ba.reshape(1, D), Wb, bb.reshape(1, D))


def _tc_layer2(h2, agg, scale, Wa, ba, Wb, bb):
    def body(scale_ref, h_ref, a0_ref, a1_ref, wa_ref, ba_ref, wb_ref, bb_ref, o_ref):
        h_blk = jnp.concatenate([h_ref[0], h_ref[1]], axis=1)
        agg_blk = jnp.concatenate([a0_ref[...], a1_ref[...]], axis=1)
        z = h_blk * scale_ref[0] + agg_blk
        o_ref[...] = _mlp(z, 1.0, wa_ref[...], ba_ref[...], wb_ref[...], bb_ref[...], False)

    return pl.pallas_call(
        body,
        grid=(N // R,),
        in_specs=[
            pl.BlockSpec(memory_space=pltpu.SMEM),
            pl.BlockSpec((NC, R, DH), lambda i: (0, i, 0)),
            *_agg_specs(),
            *_w_specs(),
        ],
        out_specs=pl.BlockSpec((R, D), lambda i: (i, 0)),
        out_shape=jax.ShapeDtypeStruct((N, D), jnp.float32),
    )(scale, h2, agg, agg, Wa, ba.reshape(1, D), Wb, bb.reshape(1, D))


def kernel(x, edge_index, eps1, W1a, b1a, W1b, b1b, eps2, W2a, b2a, W2b, b2b):
    src = jnp.pad(edge_index[0], (0, E_PAD - E)).reshape(NS, CPT, K)
    dst = jnp.pad(edge_index[1], (0, E_PAD - E),
                  constant_values=N).reshape(NS, CPT, K)
    srcx = jnp.concatenate([src, src + N], axis=0)
    s1 = (1.0 + eps1).reshape(1)
    s2 = (1.0 + eps2).reshape(1)

    x2 = jnp.concatenate([x[:, :DH], x[:, DH:]], axis=0)
    agg1 = _sc_segsum(x2, srcx, dst)
    h2 = _tc_layer1(x, agg1, s1, W1a, b1a, W1b, b1b)
    agg2 = _sc_segsum(h2.reshape(NC * N, DH), srcx, dst)
    return _tc_layer2(h2, agg2, s2, W2a, b2a, W2b, b2b)


# speedup vs baseline: 1.3126x; 1.0068x over previous
import functools

import jax
import jax.numpy as jnp
from jax import lax
from jax.experimental import pallas as pl
from jax.experimental.pallas import tpu as pltpu
from jax.experimental.pallas import tpu_sc as plsc

N = 10000
E = 320000
D = 128
DH = D // 2
NC = 2
NS = 16
K = 128
CPT = 158
EPT = CPT * K
E_PAD = NS * EPT
NBUF = 2
N_PAD = 10240
RPT = N_PAD // NS
RCH = 80


def _sc_segsum(h2, srcx, dstx):
    mesh = plsc.VectorSubcoreMesh(core_axis_name="c", subcore_axis_name="s")

    @functools.partial(
        pl.kernel,
        out_type=jax.ShapeDtypeStruct((NC * N, DH), jnp.float32),
        mesh=mesh,
        scratch_types=[
            pltpu.VMEM((CPT, K), jnp.int32),
            pltpu.VMEM((CPT, K), jnp.int32),
            pltpu.VMEM((NBUF, K, DH), jnp.float32),
            pltpu.VMEM((RCH, DH), jnp.float32),
            pltpu.VMEM_SHARED((N_PAD, DH), jnp.float32),
            [pltpu.SemaphoreType.DMA] * NBUF,
        ],
        compiler_params=pltpu.CompilerParams(use_tc_tiling_on_sc=False),
    )
    def seg_kernel(h_hbm, src_hbm, dst_hbm, out_hbm,
                   src_v, dst_v, rows_v, buf_v, acc_sh, gsems):
        cid = lax.axis_index("c")
        sid = lax.axis_index("s")

        zeros16 = jnp.zeros((16,), jnp.float32)

        def zrow(i, carry):
            for c16 in range(DH // 16):
                buf_v[i, pl.ds(c16 * 16, 16)] = zeros16
            return carry

        lax.fori_loop(0, RCH, zrow, 0)
        row0 = sid * RPT
        for r in range(RPT // RCH):
            pltpu.sync_copy(buf_v, acc_sh.at[pl.ds(row0 + r * RCH, RCH)])
        plsc.subcore_barrier()

        pltpu.sync_copy(src_hbm.at[cid * NS + sid], src_v)
        pltpu.sync_copy(dst_hbm.at[sid], dst_v)

        for b in range(NBUF):
            pltpu.async_copy(h_hbm.at[src_v.at[b]], rows_v.at[b], gsems[b])

        def step(j, b):
            pltpu.make_async_copy(h_hbm.at[src_v.at[j]], rows_v.at[b], gsems[b]).wait()
            pltpu.sync_copy(rows_v.at[b], acc_sh.at[dst_v.at[j]], add=True)

        def outer(i, carry):
            for b in range(NBUF):
                j = i * NBUF + b
                step(j, b)
                pltpu.async_copy(h_hbm.at[src_v.at[j + NBUF]], rows_v.at[b], gsems[b])
            return carry

        lax.fori_loop(0, (CPT - NBUF) // NBUF, outer, 0)
        for b in range(NBUF):
            step(CPT - NBUF + b, b)
        plsc.subcore_barrier()

        nch = jnp.minimum(RPT, N - row0) // RCH

        def wb(r, carry):
            r0 = row0 + r * RCH
            pltpu.sync_copy(acc_sh.at[pl.ds(r0, RCH)], buf_v)
            pltpu.sync_copy(buf_v, out_hbm.at[pl.ds(cid * N + r0, RCH)])
            return carry

        lax.fori_loop(0, nch, wb, 0)

    return seg_kernel(h2, srcx, dstx)


R = 2000


def _agg_specs():
    return [
        pl.BlockSpec((R, DH), lambda i: (i, 0)),
        pl.BlockSpec((R, DH), lambda i: (N // R + i, 0)),
    ]


def _w_specs():
    return [
        pl.BlockSpec((DH, D), lambda i: (0, 0)),
        pl.BlockSpec((DH, D), lambda i: (0, 0)),
        pl.BlockSpec((1, D), lambda i: (0, 0)),
        pl.BlockSpec((D, D), lambda i: (0, 0)),
        pl.BlockSpec((1, D), lambda i: (0, 0)),
    ]


def _mlp2(z0, z1, wa0, wa1, ba, wb, bb):
    z = (jnp.dot(z0, wa0, preferred_element_type=jnp.float32)
         + jnp.dot(z1, wa1, preferred_element_type=jnp.float32) + ba)
    z = jnp.maximum(z, 0.0)
    return jnp.dot(z, wb, preferred_element_type=jnp.float32) + bb


def _tc_layer1(x, agg, scale, Wa, ba, Wb, bb):
    def body(scale_ref, h_ref, a0_ref, a1_ref, wa0_ref, wa1_ref, ba_ref,
             wb_ref, bb_ref, o_ref):
        s = scale_ref[0]
        z0 = h_ref[:, :DH] * s + a0_ref[...]
        z1 = h_ref[:, DH:] * s + a1_ref[...]
        o = jnp.maximum(_mlp2(z0, z1, wa0_ref[...], wa1_ref[...], ba_ref[...],
                              wb_ref[...], bb_ref[...]), 0.0)
        o_ref[0] = o[:, :DH]
        o_ref[1] = o[:, DH:]

    return pl.pallas_call(
        body,
        grid=(N // R,),
        in_specs=[
            pl.BlockSpec(memory_space=pltpu.SMEM),
            pl.BlockSpec((R, D), lambda i: (i, 0)),
            *_agg_specs(),
            *_w_specs(),
        ],
        out_specs=pl.BlockSpec((NC, R, DH), lambda i: (0, i, 0)),
        out_shape=jax.ShapeDtypeStruct((NC, N, DH), jnp.float32),
    )(scale, x, agg, agg, Wa[:DH], Wa[DH:], ba.reshape(1, D), Wb,
      bb.reshape(1, D))


def _tc_layer2(h2, agg, scale, Wa, ba, Wb, bb):
    def body(scale_ref, h_ref, a0_ref, a1_ref, wa0_ref, wa1_ref, ba_ref,
             wb_ref, bb_ref, o_ref):
        s = scale_ref[0]
        z0 = h_ref[0] * s + a0_ref[...]
        z1 = h_ref[1] * s + a1_ref[...]
        o_ref[...] = _mlp2(z0, z1, wa0_ref[...], wa1_ref[...], ba_ref[...],
                           wb_ref[...], bb_ref[...])

    return pl.pallas_call(
        body,
        grid=(N // R,),
        in_specs=[
            pl.BlockSpec(memory_space=pltpu.SMEM),
            pl.BlockSpec((NC, R, DH), lambda i: (0, i, 0)),
            *_agg_specs(),
            *_w_specs(),
        ],
        out_specs=pl.BlockSpec((R, D), lambda i: (i, 0)),
        out_shape=jax.ShapeDtypeStruct((N, D), jnp.float32),
    )(scale, h2, agg, agg, Wa[:DH], Wa[DH:], ba.reshape(1, D), Wb,
      bb.reshape(1, D))


def kernel(x, edge_index, eps1, W1a, b1a, W1b, b1b, eps2, W2a, b2a, W2b, b2b):
    src = jnp.pad(edge_index[0], (0, E_PAD - E)).reshape(NS, CPT, K)
    dst = jnp.pad(edge_index[1], (0, E_PAD - E),
                  constant_values=N).reshape(NS, CPT, K)
    srcx = jnp.concatenate([src, src + N], axis=0)
    s1 = (1.0 + eps1).reshape(1)
    s2 = (1.0 + eps2).reshape(1)

    x2 = jnp.concatenate([x[:, :DH], x[:, DH:]], axis=0)
    agg1 = _sc_segsum(x2, srcx, dst)
    h2 = _tc_layer1(x, agg1, s1, W1a, b1a, W1b, b1b)
    agg2 = _sc_segsum(h2.reshape(NC * N, DH), srcx, dst)
    return _tc_layer2(h2, agg2, s2, W2a, b2a, W2b, b2b)


# speedup vs baseline: 1.3236x; 1.0084x over previous
import functools

import jax
import jax.numpy as jnp
from jax import lax
from jax.experimental import pallas as pl
from jax.experimental.pallas import tpu as pltpu
from jax.experimental.pallas import tpu_sc as plsc

N = 10000
E = 320000
D = 128
DH = D // 2
NC = 2
NS = 16
K = 128
CPT = 158
EPT = CPT * K
E_PAD = NS * EPT
NBUF = 4
N_PAD = 10240
RPT = N_PAD // NS
RCH = 80


def _sc_segsum(h2, srcx, dstx):
    mesh = plsc.VectorSubcoreMesh(core_axis_name="c", subcore_axis_name="s")

    @functools.partial(
        pl.kernel,
        out_type=jax.ShapeDtypeStruct((NC * N, DH), jnp.float32),
        mesh=mesh,
        scratch_types=[
            pltpu.VMEM((CPT, K), jnp.int32),
            pltpu.VMEM((CPT, K), jnp.int32),
            pltpu.VMEM((NBUF, K, DH), jnp.float32),
            pltpu.VMEM((RCH, DH), jnp.float32),
            pltpu.VMEM_SHARED((N_PAD, DH), jnp.float32),
            [pltpu.SemaphoreType.DMA] * NBUF,
            [pltpu.SemaphoreType.DMA] * NBUF,
        ],
        compiler_params=pltpu.CompilerParams(use_tc_tiling_on_sc=False),
    )
    def seg_kernel(h_hbm, src_hbm, dst_hbm, out_hbm,
                   src_v, dst_v, rows_v, buf_v, acc_sh, gsems, ssems):
        cid = lax.axis_index("c")
        sid = lax.axis_index("s")

        zeros16 = jnp.zeros((16,), jnp.float32)

        def zrow(i, carry):
            for c16 in range(DH // 16):
                buf_v[i, pl.ds(c16 * 16, 16)] = zeros16
            return carry

        lax.fori_loop(0, RCH, zrow, 0)
        row0 = sid * RPT
        for r in range(RPT // RCH):
            pltpu.sync_copy(buf_v, acc_sh.at[pl.ds(row0 + r * RCH, RCH)])
        plsc.subcore_barrier()

        pltpu.sync_copy(src_hbm.at[cid * NS + sid], src_v)
        pltpu.sync_copy(dst_hbm.at[sid], dst_v)

        def g_start(j, b):
            pltpu.async_copy(h_hbm.at[src_v.at[j]], rows_v.at[b], gsems[b])

        def g_wait(j, b):
            pltpu.make_async_copy(h_hbm.at[src_v.at[j]], rows_v.at[b], gsems[b]).wait()

        def s_start(j, b):
            pltpu.async_copy(rows_v.at[b], acc_sh.at[dst_v.at[j]], ssems[b], add=True)

        def s_wait(j, b):
            pltpu.make_async_copy(rows_v.at[b], acc_sh.at[dst_v.at[j]], ssems[b]).wait()

        g_start(0, 0)
        g_start(1, 1)
        for j in (0, 1):
            g_wait(j, j)
            s_start(j, j)
            g_start(j + 2, j + 2)

        def outer(i, carry):
            for b in range(4):
                j = i * 4 + 2 + b
                bj = (2 + b) % 4
                g_wait(j, bj)
                s_start(j, bj)
                s_wait(j - 2, b)
                g_start(j + 2, b)
            return carry

        lax.fori_loop(0, (CPT - 6) // 4, outer, 0)
        for j in (CPT - 4, CPT - 3):
            g_wait(j, j % 4)
            s_start(j, j % 4)
            s_wait(j - 2, (j + 2) % 4)
            g_start(j + 2, (j + 2) % 4)
        for j in (CPT - 2, CPT - 1):
            g_wait(j, j % 4)
            s_start(j, j % 4)
        for j in range(CPT - 4, CPT):
            s_wait(j, j % 4)
        plsc.subcore_barrier()

        nch = jnp.minimum(RPT, N - row0) // RCH

        def wb(r, carry):
            r0 = row0 + r * RCH
            pltpu.sync_copy(acc_sh.at[pl.ds(r0, RCH)], buf_v)
            pltpu.sync_copy(buf_v, out_hbm.at[pl.ds(cid * N + r0, RCH)])
            return carry

        lax.fori_loop(0, nch, wb, 0)

    return seg_kernel(h2, srcx, dstx)


R = 2000


def _agg_specs():
    return [
        pl.BlockSpec((R, DH), lambda i: (i, 0)),
        pl.BlockSpec((R, DH), lambda i: (N // R + i, 0)),
    ]


def _w_specs():
    return [
        pl.BlockSpec((DH, D), lambda i: (0, 0)),
        pl.BlockSpec((DH, D), lambda i: (0, 0)),
        pl.BlockSpec((1, D), lambda i: (0, 0)),
        pl.BlockSpec((D, D), lambda i: (0, 0)),
        pl.BlockSpec((1, D), lambda i: (0, 0)),
    ]


def _mlp2(z0, z1, wa0, wa1, ba, wb, bb):
    z = (jnp.dot(z0, wa0, preferred_element_type=jnp.float32)
         + jnp.dot(z1, wa1, preferred_element_type=jnp.float32) + ba)
    z = jnp.maximum(z, 0.0)
    return jnp.dot(z, wb, preferred_element_type=jnp.float32) + bb


def _tc_layer1(x, agg, scale, Wa, ba, Wb, bb):
    def body(scale_ref, h_ref, a0_ref, a1_ref, wa0_ref, wa1_ref, ba_ref,
             wb_ref, bb_ref, o_ref):
        s = scale_ref[0]
        z0 = h_ref[:, :DH] * s + a0_ref[...]
        z1 = h_ref[:, DH:] * s + a1_ref[...]
        o = jnp.maximum(_mlp2(z0, z1, wa0_ref[...], wa1_ref[...], ba_ref[...],
                              wb_ref[...], bb_ref[...]), 0.0)
        o_ref[0] = o[:, :DH]
        o_ref[1] = o[:, DH:]

    return pl.pallas_call(
        body,
        grid=(N // R,),
        in_specs=[
            pl.BlockSpec(memory_space=pltpu.SMEM),
            pl.BlockSpec((R, D), lambda i: (i, 0)),
            *_agg_specs(),
            *_w_specs(),
        ],
        out_specs=pl.BlockSpec((NC, R, DH), lambda i: (0, i, 0)),
        out_shape=jax.ShapeDtypeStruct((NC, N, DH), jnp.float32),
    )(scale, x, agg, agg, Wa[:DH], Wa[DH:], ba.reshape(1, D), Wb,
      bb.reshape(1, D))


def _tc_layer2(h2, agg, scale, Wa, ba, Wb, bb):
    def body(scale_ref, h_ref, a0_ref, a1_ref, wa0_ref, wa1_ref, ba_ref,
             wb_ref, bb_ref, o_ref):
        s = scale_ref[0]
        z0 = h_ref[0] * s + a0_ref[...]
        z1 = h_ref[1] * s + a1_ref[...]
        o_ref[...] = _mlp2(z0, z1, wa0_ref[...], wa1_ref[...], ba_ref[...],
                           wb_ref[...], bb_ref[...])

    return pl.pallas_call(
        body,
        grid=(N // R,),
        in_specs=[
            pl.BlockSpec(memory_space=pltpu.SMEM),
            pl.BlockSpec((NC, R, DH), lambda i: (0, i, 0)),
            *_agg_specs(),
            *_w_specs(),
        ],
        out_specs=pl.BlockSpec((R, D), lambda i: (i, 0)),
        out_shape=jax.ShapeDtypeStruct((N, D), jnp.float32),
    )(scale, h2, agg, agg, Wa[:DH], Wa[DH:], ba.reshape(1, D), Wb,
      bb.reshape(1, D))


def kernel(x, edge_index, eps1, W1a, b1a, W1b, b1b, eps2, W2a, b2a, W2b, b2b):
    src = jnp.pad(edge_index[0], (0, E_PAD - E)).reshape(NS, CPT, K)
    dst = jnp.pad(edge_index[1], (0, E_PAD - E),
                  constant_values=N).reshape(NS, CPT, K)
    srcx = jnp.concatenate([src, src + N], axis=0)
    s1 = (1.0 + eps1).reshape(1)
    s2 = (1.0 + eps2).reshape(1)

    x2 = jnp.concatenate([x[:, :DH], x[:, DH:]], axis=0)
    agg1 = _sc_segsum(x2, srcx, dst)
    h2 = _tc_layer1(x, agg1, s1, W1a, b1a, W1b, b1b)
    agg2 = _sc_segsum(h2.reshape(NC * N, DH), srcx, dst)
    return _tc_layer2(h2, agg2, s2, W2a, b2a, W2b, b2b)
